# baseline (speedup 1.0000x reference)
import jax
import jax.numpy as jnp
from jax import lax
from jax.experimental import pallas as pl
from jax.experimental.pallas import tpu as pltpu
from jax.experimental.pallas import tpu_sc as plsc

BATCH = 16384
N_FIELDS = 26
NUM_FACTORS = 16
NUM_INPUTS = 1000012

NC = 2
NS = 16
NW = NC * NS

V_PAD = 1000016
SHARE = 31256
RC = 128
N_RCH = 246

ROWS_PER_W = BATCH // NW
IDX_PER_W = ROWS_PER_W * N_FIELDS
GATHER_W = 128
IDX_ROWS_PER_W = IDX_PER_W // GATHER_W
CHUNK_ROWS = 64
CHUNK_IDX = CHUNK_ROWS * N_FIELDS
DMAS_PER_CHUNK = CHUNK_IDX // GATHER_W
N_CHUNKS = ROWS_PER_W // CHUNK_ROWS


def _relayout_body(emb_hbm, fc_hbm, elin_hbm, flin_hbm,
                   ebuf, fbuf, elin, flin, sem_r, sem_w):
    wid = lax.axis_index("s") * NC + lax.axis_index("c")
    start = wid * SHARE
    limit = lax.min(start + SHARE, V_PAD) - RC
    lane_ids = lax.iota(jnp.int32, NUM_FACTORS)
    zeros16 = jnp.zeros((NUM_FACTORS,), jnp.int32)
    fzeros16 = jnp.zeros((NUM_FACTORS,), jnp.float32)

    def r0_of(ch):
        return lax.min(start + RC * ch, limit)

    def read_descs(p, r0):
        return (
            pltpu.make_async_copy(emb_hbm.at[pl.ds(r0, RC), :], ebuf.at[p],
                                  sem_r),
            pltpu.make_async_copy(fc_hbm.at[pl.ds(r0, RC), :], fbuf.at[p],
                                  sem_r),
        )

    def write_descs(p, r0):
        return (
            pltpu.make_async_copy(elin.at[p],
                                  elin_hbm.at[pl.ds(r0 * NUM_FACTORS,
                                                    RC * NUM_FACTORS)],
                                  sem_w),
            pltpu.make_async_copy(flin.at[p], flin_hbm.at[pl.ds(r0, RC)],
                                  sem_w),
        )

    for p in (0, 1):
        for d in read_descs(p, r0_of(p)):
            d.start()

    def body2(i, carry):
        for p in (0, 1):
            ch = 2 * i + p
            r0 = r0_of(ch)
            for d in read_descs(p, r0):
                d.wait()

            @pl.when(i > 0)
            def _wait_prev_writes():
                for d in write_descs(p, r0_of(ch - 2)):
                    d.wait()

            def compact_row(r, facc):
                elin[p, pl.ds(r * NUM_FACTORS, NUM_FACTORS)] = \
                    ebuf[p, r, pl.ds(0, NUM_FACTORS)]
                fv = fbuf[p, r, pl.ds(0, NUM_FACTORS)]
                fb = fv.at[zeros16].get(mode="promise_in_bounds")
                lane = r % NUM_FACTORS
                facc = jnp.where(lane_ids == lane, fb, facc)

                @pl.when(lane == NUM_FACTORS - 1)
                def _flush_fc():
                    flin[p, pl.ds(r - (NUM_FACTORS - 1), NUM_FACTORS)] = facc

                return facc

            lax.fori_loop(0, RC, compact_row, fzeros16)

            for d in write_descs(p, r0):
                d.start()
            for d in read_descs(p, r0_of(ch + 2)):
                d.start()
        return carry

    lax.fori_loop(0, N_RCH // 2, body2, 0)

    for p in (0, 1):
        for d in read_descs(p, r0_of(N_RCH + p)):
            d.wait()
        for d in write_descs(p, r0_of(N_RCH - 2 + p)):
            d.wait()


def _fm_body(x_hbm, emb_hbm, fc_hbm, out_hbm, fcout_hbm,
             idx_v, emb_buf, fc_buf, out_v, fcvec_v, sem):
    wid = lax.axis_index("s") * NC + lax.axis_index("c")

    pltpu.sync_copy(x_hbm.at[pl.ds(wid * IDX_ROWS_PER_W, IDX_ROWS_PER_W), :],
                    idx_v)

    def start_chunk(g, ebuf, fbuf):
        handles = []
        for j in range(DMAS_PER_CHUNK):
            isl = idx_v.at[g * DMAS_PER_CHUNK + j]
            handles.append(pltpu.async_copy(
                emb_hbm.at[isl], ebuf.at[pl.ds(j * GATHER_W, GATHER_W), :],
                sem))
            handles.append(pltpu.async_copy(
                fc_hbm.at[isl], fbuf.at[pl.ds(j * GATHER_W, GATHER_W)], sem))
        return handles

    lane_ids = lax.iota(jnp.int32, NUM_FACTORS)

    def hsum(v):
        for k in (8, 4, 2, 1):
            v = v + v.at[lane_ids ^ k].get(mode="promise_in_bounds")
        return v

    def compute_chunk(g, ebuf, fbuf, fc_acc):
        out_base = g * CHUNK_ROWS

        def row_body(r, acc):
            rbase = r * N_FIELDS
            s = ebuf[rbase, :]
            sq = s * s
            for j in range(1, N_FIELDS):
                v = ebuf[rbase + j, :]
                s = s + v
                sq = sq + v * v
            inter = hsum(s * s - sq)
            lane = r % NUM_FACTORS
            acc = jnp.where(lane_ids == lane, 0.5 * inter, acc)

            @pl.when(lane == NUM_FACTORS - 1)
            def _flush():
                out_v[pl.ds(out_base + r - (NUM_FACTORS - 1), NUM_FACTORS)] \
                    = acc

            return acc

        lax.fori_loop(0, CHUNK_ROWS, row_body,
                      jnp.zeros((NUM_FACTORS,), jnp.float32))

        return lax.fori_loop(
            0, CHUNK_IDX // NUM_FACTORS,
            lambda k, a: a + fbuf[pl.ds(k * NUM_FACTORS, NUM_FACTORS)],
            fc_acc)

    fc_acc = jnp.zeros((NUM_FACTORS,), jnp.float32)
    handles = start_chunk(0, emb_buf.at[0], fc_buf.at[0])
    for g in range(N_CHUNKS):
        p = g % 2
        for h in handles:
            h.wait()
        if g + 1 < N_CHUNKS:
            handles = start_chunk(g + 1, emb_buf.at[1 - p], fc_buf.at[1 - p])
        else:
            handles = []
        fc_acc = compute_chunk(g, emb_buf.at[p], fc_buf.at[p], fc_acc)

    fcvec_v[:] = fc_acc
    pltpu.sync_copy(out_v, out_hbm.at[pl.ds(wid * ROWS_PER_W, ROWS_PER_W)])
    pltpu.sync_copy(fcvec_v, fcout_hbm.at[wid])


def kernel(X, emb_table, fc_table, dense_W, dense_b):
    x_flat = X.astype(jnp.int32).reshape(BATCH * N_FIELDS // GATHER_W,
                                         GATHER_W)

    mesh = plsc.VectorSubcoreMesh(core_axis_name="c", subcore_axis_name="s")

    relayout = pl.kernel(
        _relayout_body,
        mesh=mesh,
        compiler_params=pltpu.CompilerParams(use_tc_tiling_on_sc=True),
        out_type=[
            jax.ShapeDtypeStruct((V_PAD * NUM_FACTORS,), jnp.float32),
            jax.ShapeDtypeStruct((V_PAD,), jnp.float32),
        ],
        scratch_types=[
            pltpu.VMEM((2, RC, NUM_FACTORS), jnp.float32),
            pltpu.VMEM((2, RC, 1), jnp.float32),
            pltpu.VMEM((2, RC * NUM_FACTORS), jnp.float32),
            pltpu.VMEM((2, RC), jnp.float32),
            pltpu.SemaphoreType.DMA,
            pltpu.SemaphoreType.DMA,
        ],
    )
    emb_lin, fc_lin = relayout(emb_table, fc_table)

    fm = pl.kernel(
        _fm_body,
        mesh=mesh,
        compiler_params=pltpu.CompilerParams(use_tc_tiling_on_sc=False),
        out_type=[
            jax.ShapeDtypeStruct((BATCH,), jnp.float32),
            jax.ShapeDtypeStruct((NW, NUM_FACTORS), jnp.float32),
        ],
        scratch_types=[
            pltpu.VMEM((IDX_ROWS_PER_W, GATHER_W), jnp.int32),
            pltpu.VMEM((2, CHUNK_IDX, NUM_FACTORS), jnp.float32),
            pltpu.VMEM((2, CHUNK_IDX), jnp.float32),
            pltpu.VMEM((ROWS_PER_W,), jnp.float32),
            pltpu.VMEM((NUM_FACTORS,), jnp.float32),
            pltpu.SemaphoreType.DMA,
        ],
    )
    inter_half, fc_parts = fm(x_flat, emb_lin.reshape(V_PAD, NUM_FACTORS),
                              fc_lin)

    linear_term = (dense_W[0, 0] * jnp.sum(fc_parts)
                   + dense_b[0] * (BATCH * N_FIELDS))
    return inter_half[:, None] + linear_term


# speedup vs baseline: 1.8705x; 1.8705x over previous
import jax
import jax.numpy as jnp
from jax import lax
from jax.experimental import pallas as pl
from jax.experimental.pallas import tpu as pltpu
from jax.experimental.pallas import tpu_sc as plsc

BATCH = 16384
N_FIELDS = 26
NUM_FACTORS = 16

NC = 2
NS = 16
NW = NC * NS
ROWS_PER_W = BATCH // NW
IDX_PER_W = ROWS_PER_W * N_FIELDS
GATHER_W = 128
IDX_ROWS_PER_W = IDX_PER_W // GATHER_W
CHUNK_ROWS = 64
CHUNK_IDX = CHUNK_ROWS * N_FIELDS
DMAS_PER_CHUNK = CHUNK_IDX // GATHER_W
N_CHUNKS = ROWS_PER_W // CHUNK_ROWS


def _fm_body(x_hbm, emb_hbm, fc_hbm, out_hbm, fcout_hbm,
             idx_v, emb_buf, fc_buf, out_v, fcvec_v, sem):
    wid = lax.axis_index("s") * NC + lax.axis_index("c")

    pltpu.sync_copy(x_hbm.at[pl.ds(wid * IDX_ROWS_PER_W, IDX_ROWS_PER_W), :],
                    idx_v)

    def start_chunk(g, ebuf, fbuf):
        handles = []
        for j in range(DMAS_PER_CHUNK):
            isl = idx_v.at[g * DMAS_PER_CHUNK + j]
            handles.append(pltpu.async_copy(
                emb_hbm.at[isl], ebuf.at[pl.ds(j * GATHER_W, GATHER_W), :], sem))
            handles.append(pltpu.async_copy(
                fc_hbm.at[isl], fbuf.at[pl.ds(j * GATHER_W, GATHER_W)], sem))
        return handles

    lane_ids = lax.iota(jnp.int32, NUM_FACTORS)

    def hsum(v):
        for k in (8, 4, 2, 1):
            v = v + v.at[lane_ids ^ k].get(mode="promise_in_bounds")
        return v

    def compute_chunk(g, ebuf, fbuf, fc_acc):
        out_base = g * CHUNK_ROWS

        def pair_body(i, acc):
            r = 2 * i
            inters = []
            for dr in (0, 1):
                rbase = (r + dr) * N_FIELDS
                s = ebuf[rbase, :]
                sq = s * s
                for j in range(1, N_FIELDS):
                    v = ebuf[rbase + j, :]
                    s = s + v
                    sq = sq + v * v
                inters.append(s * s - sq)
            h0, h1 = hsum(inters[0]), hsum(inters[1])
            lane = r % NUM_FACTORS
            acc = jnp.where(lane_ids == lane, 0.5 * h0, acc)
            acc = jnp.where(lane_ids == lane + 1, 0.5 * h1, acc)

            @pl.when(lane == NUM_FACTORS - 2)
            def _flush():
                out_v[pl.ds(out_base + r - (NUM_FACTORS - 2), NUM_FACTORS)] = acc

            return acc

        lax.fori_loop(0, CHUNK_ROWS // 2, pair_body,
                      jnp.zeros((NUM_FACTORS,), jnp.float32))

        return lax.fori_loop(
            0, CHUNK_IDX // NUM_FACTORS,
            lambda k, a: a + fbuf[pl.ds(k * NUM_FACTORS, NUM_FACTORS)],
            fc_acc)

    fc_acc = jnp.zeros((NUM_FACTORS,), jnp.float32)
    handles = start_chunk(0, emb_buf.at[0], fc_buf.at[0])
    for g in range(N_CHUNKS):
        p = g % 2
        for h in handles:
            h.wait()
        if g + 1 < N_CHUNKS:
            handles = start_chunk(g + 1, emb_buf.at[1 - p], fc_buf.at[1 - p])
        else:
            handles = []
        fc_acc = compute_chunk(g, emb_buf.at[p], fc_buf.at[p], fc_acc)

    fcvec_v[:] = fc_acc
    pltpu.sync_copy(out_v, out_hbm.at[pl.ds(wid * ROWS_PER_W, ROWS_PER_W)])
    pltpu.sync_copy(fcvec_v, fcout_hbm.at[wid])


def kernel(X, emb_table, fc_table, dense_W, dense_b):
    x_flat = X.astype(jnp.int32).reshape(BATCH * N_FIELDS // GATHER_W, GATHER_W)
    fc_flat = fc_table.reshape(-1)

    mesh = plsc.VectorSubcoreMesh(core_axis_name="c", subcore_axis_name="s")
    fm = pl.kernel(
        _fm_body,
        mesh=mesh,
        compiler_params=pltpu.CompilerParams(use_tc_tiling_on_sc=False),
        out_type=[
            jax.ShapeDtypeStruct((BATCH,), jnp.float32),
            jax.ShapeDtypeStruct((NW, NUM_FACTORS), jnp.float32),
        ],
        scratch_types=[
            pltpu.VMEM((IDX_ROWS_PER_W, GATHER_W), jnp.int32),
            pltpu.VMEM((2, CHUNK_IDX, NUM_FACTORS), jnp.float32),
            pltpu.VMEM((2, CHUNK_IDX), jnp.float32),
            pltpu.VMEM((ROWS_PER_W,), jnp.float32),
            pltpu.VMEM((NUM_FACTORS,), jnp.float32),
            pltpu.SemaphoreType.DMA,
        ],
    )
    inter_half, fc_parts = fm(x_flat, emb_table, fc_flat)

    linear_term = (dense_W[0, 0] * jnp.sum(fc_parts)
                   + dense_b[0] * (BATCH * N_FIELDS))
    return inter_half[:, None] + linear_term
